# baseline (speedup 1.0000x reference)
import jax
import jax.numpy as jnp
from jax import lax
from jax.experimental import pallas as pl
from jax.experimental.pallas import tpu as pltpu, tpu_sc as plsc

BATCH = 16384
POSES = 64
N_IMG = 100
H = 256
W = 256
GRID = H * W

NW = 32
SPW = BATCH // NW
WPW = SPW * 3


def _u32(x):
    return jnp.uint32(x)


def _threefry2x32(k0, k1, x0, x1):
    ks2 = k0 ^ k1 ^ _u32(0x1BD11BDA)
    ks = (k0, k1, ks2)
    rot_a = (13, 15, 26, 6)
    rot_b = (17, 29, 16, 24)
    x0 = x0 + k0
    x1 = x1 + k1
    for i in range(5):
        for r in (rot_a if i % 2 == 0 else rot_b):
            x0 = x0 + x1
            x1 = ((x1 << _u32(r)) | (x1 >> _u32(32 - r))) ^ x0
        x0 = x0 + ks[(i + 1) % 3]
        x1 = x1 + ks[(i + 2) % 3] + _u32(i + 1)
    return x0, x1


def _dg(vec, idx):
    return vec.at[idx].get(mode="promise_in_bounds")


def _splat(vec_u32, lane):
    idx = jnp.full((16,), lane, dtype=jnp.int32)
    v = plsc.bitcast(vec_u32, jnp.int32)
    return plsc.bitcast(_dg(v, idx), jnp.uint32)


def _sc_body(kd_hbm, img_hbm, org_hbm, dir_hbm,
             out_org, out_dir, out_key, out_pix,
             kd_v, gidx_v, pidx_v, key_v,
             org_w, dir_w, pix_w, sem):
    wid = lax.axis_index("s") * 2 + lax.axis_index("c")
    base = wid * SPW

    pltpu.sync_copy(kd_hbm, kd_v)
    zero16 = jnp.zeros((16,), jnp.uint32)
    iota16 = lax.iota(jnp.int32, 16).astype(jnp.uint32)

    kdvec = plsc.bitcast(kd_v[...], jnp.uint32)
    k0 = _splat(kdvec, 0)
    k1 = _splat(kdvec, 1)

    s0, s1 = _threefry2x32(k0, k1, zero16, iota16)
    pk0, pk1 = _splat(s0, 0), _splat(s1, 0)
    dk0, dk1 = _splat(s0, 1), _splat(s1, 1)
    rk0, rk1 = _splat(s0, 2), _splat(s1, 2)

    keyvec = jnp.where(iota16 == _u32(0), rk0, rk1)
    key_v[...] = plsc.bitcast(keyvec, jnp.int32)

    @pl.when(wid == 0)
    def _():
        pltpu.sync_copy(key_v, out_key)

    t0, t1 = _threefry2x32(pk0, pk1, zero16, iota16)
    pA0, pA1 = _splat(t0, 0), _splat(t1, 0)
    pB0, pB1 = _splat(t0, 1), _splat(t1, 1)

    u0, u1 = _threefry2x32(dk0, dk1, zero16, iota16)
    dB0, dB1 = _splat(u0, 1), _splat(u1, 1)

    span = _u32(100)
    cnt = jnp.full((16,), 2 * wid, jnp.int32).astype(jnp.uint32) + iota16
    h0, h1 = _threefry2x32(pA0, pA1, zero16, cnt)
    l0, l1 = _threefry2x32(pB0, pB1, zero16, cnt)
    posevec = (((h0 ^ h1) % span) * _u32(96) + ((l0 ^ l1) % span)) % span
    p_splat = (plsc.bitcast(_splat(posevec, 0), jnp.int32),
               plsc.bitcast(_splat(posevec, 1), jnp.int32))

    ii = lax.iota(jnp.int32, 16)
    div3 = tuple(((ii + 16 * r) * 21846) >> 16 for r in range(3))
    mod3 = tuple((ii + 16 * r) - 3 * d for r, d in enumerate(div3))

    for c in range(32):
        pose = p_splat[c // 16]
        i0 = base + c * 16
        cnt = jnp.full((16,), i0, jnp.int32).astype(jnp.uint32) + iota16
        d0, d1 = _threefry2x32(dB0, dB1, zero16, cnt)
        dflat = plsc.bitcast((d0 ^ d1) & _u32(0xFFFF), jnp.int32)
        g3 = ((pose << 16) + dflat) * 3
        for r in range(3):
            w0 = c * 48 + r * 16
            gidx_v[pl.ds(w0, 16)] = _dg(g3, div3[r]) + mod3[r]

    for half in range(2):
        p3 = p_splat[half] * 3

        def org_chunk(c2, carry, p3=p3, half=half):
            for r in range(3):
                pidx_v[pl.ds(half * 768 + c2 * 48 + r * 16, 16)] = p3 + mod3[r]
            return carry

        lax.fori_loop(0, 16, org_chunk, 0, unroll=False)

    copies = []
    for j in range(WPW // 128):
        win = pl.ds(j * 128, 128)
        copies.append(pltpu.async_copy(
            img_hbm.at[gidx_v.at[win]], pix_w.at[win], sem))
        copies.append(pltpu.async_copy(
            dir_hbm.at[gidx_v.at[win]], dir_w.at[win], sem))
        copies.append(pltpu.async_copy(
            org_hbm.at[pidx_v.at[win]], org_w.at[win], sem))
    for c in copies:
        c.wait()

    out = pl.ds(wid * WPW, WPW)
    pltpu.sync_copy(org_w, out_org.at[out])
    pltpu.sync_copy(dir_w, out_dir.at[out])
    pltpu.sync_copy(pix_w, out_pix.at[out])


@jax.jit
def _sampler(kd, img_flat, org_flat, dir_flat):
    mesh = plsc.VectorSubcoreMesh(core_axis_name="c", subcore_axis_name="s")
    f32 = jnp.float32
    run = pl.kernel(
        _sc_body,
        mesh=mesh,
        out_type=(
            jax.ShapeDtypeStruct((BATCH * 3,), f32),
            jax.ShapeDtypeStruct((BATCH * 3,), f32),
            jax.ShapeDtypeStruct((16,), jnp.int32),
            jax.ShapeDtypeStruct((BATCH * 3,), f32),
        ),
        scratch_types=[
            pltpu.VMEM((16,), jnp.int32),
            pltpu.VMEM((WPW,), jnp.int32),
            pltpu.VMEM((WPW,), jnp.int32),
            pltpu.VMEM((16,), jnp.int32),
            pltpu.VMEM((WPW,), f32),
            pltpu.VMEM((WPW,), f32),
            pltpu.VMEM((WPW,), f32),
            pltpu.SemaphoreType.DMA,
        ],
    )
    return run(kd, img_flat, org_flat, dir_flat)


def kernel(key, images, ray_origins, ray_directions):
    kd = lax.bitcast_convert_type(jax.random.key_data(key), jnp.int32)
    kd = jnp.concatenate([kd, jnp.zeros((14,), jnp.int32)])
    img_flat = images.reshape(-1)
    dir_flat = ray_directions.reshape(-1)
    org_flat = ray_origins.reshape(-1)
    origins, dirs, keyvec, pixels = _sampler(kd, img_flat, org_flat, dir_flat)
    return (origins.reshape(BATCH, 3), dirs.reshape(BATCH, 3),
            lax.bitcast_convert_type(keyvec[:2], jnp.uint32),
            pixels.reshape(BATCH, 3))


# speedup vs baseline: 52.5691x; 52.5691x over previous
import jax
import jax.numpy as jnp
from jax import lax
from jax.experimental import pallas as pl
from jax.experimental.pallas import tpu as pltpu, tpu_sc as plsc

BATCH = 16384
N_IMG = 100
H = 256
W = 256
GRID = H * W

NW = 32
SPW = BATCH // NW
WPW = SPW * 3


def _u32(x):
    return jnp.uint32(x)


def _threefry2x32(k0, k1, x0, x1):
    ks2 = k0 ^ k1 ^ _u32(0x1BD11BDA)
    ks = (k0, k1, ks2)
    rot_a = (13, 15, 26, 6)
    rot_b = (17, 29, 16, 24)
    x0 = x0 + k0
    x1 = x1 + k1
    for i in range(5):
        for r in (rot_a if i % 2 == 0 else rot_b):
            x0 = x0 + x1
            x1 = ((x1 << _u32(r)) | (x1 >> _u32(32 - r))) ^ x0
        x0 = x0 + ks[(i + 1) % 3]
        x1 = x1 + ks[(i + 2) % 3] + _u32(i + 1)
    return x0, x1


def _splat(vec_u32, lane):
    idx = jnp.full((16,), lane, dtype=jnp.int32)
    v = plsc.bitcast(vec_u32, jnp.int32)
    return plsc.bitcast(v.at[idx].get(mode="promise_in_bounds"), jnp.uint32)


def _sc_body(kd_hbm, img_hbm, org_hbm, dir_hbm,
             out_org, out_dir, out_key, out_pix,
             kd_v, gidx_v, pidx_v, key_v,
             org_w, dir_w, pix_w, sem):
    wid = lax.axis_index("s") * 2 + lax.axis_index("c")
    base = wid * SPW

    pltpu.sync_copy(kd_hbm, kd_v)
    zero16 = jnp.zeros((16,), jnp.uint32)
    iota16 = lax.iota(jnp.int32, 16).astype(jnp.uint32)

    kdvec = plsc.bitcast(kd_v[...], jnp.uint32)
    k0 = _splat(kdvec, 0)
    k1 = _splat(kdvec, 1)

    s0, s1 = _threefry2x32(k0, k1, zero16, iota16)
    pk0, pk1 = _splat(s0, 0), _splat(s1, 0)
    dk0, dk1 = _splat(s0, 1), _splat(s1, 1)
    rk0, rk1 = _splat(s0, 2), _splat(s1, 2)

    keyvec = jnp.where(iota16 == _u32(0), rk0, rk1)
    key_v[...] = plsc.bitcast(keyvec, jnp.int32)

    @pl.when(wid == 0)
    def _():
        pltpu.sync_copy(key_v, out_key)

    t0, t1 = _threefry2x32(pk0, pk1, zero16, iota16)
    pA0, pA1 = _splat(t0, 0), _splat(t1, 0)
    pB0, pB1 = _splat(t0, 1), _splat(t1, 1)

    u0, u1 = _threefry2x32(dk0, dk1, zero16, iota16)
    dB0, dB1 = _splat(u0, 1), _splat(u1, 1)

    span = _u32(100)
    cnt = jnp.full((16,), 2 * wid, jnp.int32).astype(jnp.uint32) + iota16
    h0, h1 = _threefry2x32(pA0, pA1, zero16, cnt)
    l0, l1 = _threefry2x32(pB0, pB1, zero16, cnt)
    posevec = (((h0 ^ h1) % span) * _u32(96) + ((l0 ^ l1) % span)) % span
    p_splat = (plsc.bitcast(_splat(posevec, 0), jnp.int32),
               plsc.bitcast(_splat(posevec, 1), jnp.int32))

    for half in range(2):
        pose = p_splat[half]
        gbase = tuple((3 * pose + k) << 16 for k in range(3))
        pbase = tuple(pose + 100 * k for k in range(3))

        def dir_chunk(c2, carry, half=half, gbase=gbase, pbase=pbase):
            i0 = base + half * 256 + c2 * 16
            cnt = jnp.full((16,), i0, jnp.int32).astype(jnp.uint32) + iota16
            d0, d1 = _threefry2x32(dB0, dB1, zero16, cnt)
            dflat = plsc.bitcast((d0 ^ d1) & _u32(0xFFFF), jnp.int32)
            for k in range(3):
                w0 = k * 512 + half * 256 + c2 * 16
                gidx_v[pl.ds(w0, 16)] = gbase[k] + dflat
                pidx_v[pl.ds(w0, 16)] = pbase[k]
            return carry

        lax.fori_loop(0, 16, dir_chunk, 0, unroll=False)

    copies = []
    for j in range(WPW // 128):
        win = pl.ds(j * 128, 128)
        copies.append(pltpu.async_copy(
            img_hbm.at[gidx_v.at[win]], pix_w.at[win], sem))
        copies.append(pltpu.async_copy(
            dir_hbm.at[gidx_v.at[win]], dir_w.at[win], sem))
        copies.append(pltpu.async_copy(
            org_hbm.at[pidx_v.at[win]], org_w.at[win], sem))
    for c in copies:
        c.wait()

    for k in range(3):
        src = pl.ds(k * 512, SPW)
        dst = pl.ds(k * BATCH + base, SPW)
        pltpu.sync_copy(org_w.at[src], out_org.at[dst])
        pltpu.sync_copy(dir_w.at[src], out_dir.at[dst])
        pltpu.sync_copy(pix_w.at[src], out_pix.at[dst])


@jax.jit
def _sampler(kd, images, ray_origins, ray_directions):
    img_p = images.transpose(0, 3, 1, 2).reshape(-1)
    dir_p = ray_directions.transpose(0, 3, 1, 2).reshape(-1)
    org_p = ray_origins.T.reshape(-1)

    mesh = plsc.VectorSubcoreMesh(core_axis_name="c", subcore_axis_name="s")
    f32 = jnp.float32
    run = pl.kernel(
        _sc_body,
        mesh=mesh,
        out_type=(
            jax.ShapeDtypeStruct((3 * BATCH,), f32),
            jax.ShapeDtypeStruct((3 * BATCH,), f32),
            jax.ShapeDtypeStruct((16,), jnp.int32),
            jax.ShapeDtypeStruct((3 * BATCH,), f32),
        ),
        scratch_types=[
            pltpu.VMEM((16,), jnp.int32),
            pltpu.VMEM((WPW,), jnp.int32),
            pltpu.VMEM((WPW,), jnp.int32),
            pltpu.VMEM((16,), jnp.int32),
            pltpu.VMEM((WPW,), f32),
            pltpu.VMEM((WPW,), f32),
            pltpu.VMEM((WPW,), f32),
            pltpu.SemaphoreType.DMA,
        ],
    )
    org_t, dir_t, keyvec, pix_t = run(kd, img_p, org_p, dir_p)
    return (org_t.reshape(3, BATCH).T, dir_t.reshape(3, BATCH).T,
            lax.bitcast_convert_type(keyvec[:2], jnp.uint32),
            pix_t.reshape(3, BATCH).T)


def kernel(key, images, ray_origins, ray_directions):
    kd = lax.bitcast_convert_type(jax.random.key_data(key), jnp.int32)
    kd = jnp.concatenate([kd, jnp.zeros((14,), jnp.int32)])
    return _sampler(kd, images, ray_origins, ray_directions)
